# baseline (speedup 1.0000x reference)
import functools

import jax
import jax.numpy as jnp
from jax.experimental import pallas as pl
from jax.experimental.pallas import tpu as pltpu

_ROWS, _COLS = 16384, 50
_NCHUNKS = 4


def _copy_body(x_ref, o_ref, *sems):
    rows = _ROWS // _NCHUNKS
    copies = [
        pltpu.make_async_copy(
            x_ref.at[pl.ds(k * rows, rows)],
            o_ref.at[pl.ds(k * rows, rows)],
            sems[k],
        )
        for k in range(_NCHUNKS)
    ]
    for c in copies:
        c.start()
    for c in copies:
        c.wait()


@functools.cache
def _make_copy_kernel():
    return pl.pallas_call(
        _copy_body,
        in_specs=[pl.BlockSpec(memory_space=pl.ANY)],
        out_specs=pl.BlockSpec(memory_space=pl.ANY),
        out_shape=jax.ShapeDtypeStruct((_ROWS, _COLS), jnp.float32),
        scratch_shapes=[pltpu.SemaphoreType.DMA] * _NCHUNKS,
    )


def kernel(candidate_titles):
    return _make_copy_kernel()(candidate_titles)


# speedup vs baseline: 1.0055x; 1.0055x over previous
import functools

import jax
import jax.numpy as jnp
from jax.experimental import pallas as pl
from jax.experimental.pallas import tpu as pltpu

_ROWS, _COLS = 16384, 50
_NCHUNKS = 4


def _copy_body(x_ref, o_ref, sem):
    c = pltpu.make_async_copy(x_ref, o_ref, sem)
    c.start()
    c.wait()


@functools.cache
def _make_copy_kernel():
    return pl.pallas_call(
        _copy_body,
        in_specs=[pl.BlockSpec(memory_space=pl.ANY)],
        out_specs=pl.BlockSpec(memory_space=pl.ANY),
        out_shape=jax.ShapeDtypeStruct((_ROWS, _COLS), jnp.float32),
        scratch_shapes=[pltpu.SemaphoreType.DMA],
    )


def kernel(candidate_titles):
    return _make_copy_kernel()(candidate_titles)


# speedup vs baseline: 39.8978x; 39.6803x over previous
import functools

import jax
import jax.numpy as jnp
from jax.experimental import pallas as pl

_ROWS, _COLS = 16384, 50
_BLOCK = 2048


def _copy_body(x_ref, o_ref):
    o_ref[...] = x_ref[...]


@functools.cache
def _make_copy_kernel():
    grid = _ROWS // _BLOCK
    return pl.pallas_call(
        _copy_body,
        grid=(grid,),
        in_specs=[pl.BlockSpec((_COLS, _BLOCK), lambda i: (0, i))],
        out_specs=pl.BlockSpec((_COLS, _BLOCK), lambda i: (0, i)),
        out_shape=jax.ShapeDtypeStruct((_COLS, _ROWS), jnp.float32),
    )


def kernel(candidate_titles):
    return _make_copy_kernel()(candidate_titles.T).T
